# baseline (speedup 1.0000x reference)
import functools

import jax
import jax.numpy as jnp
from jax import lax
from jax.experimental import pallas as pl
from jax.experimental.pallas import tpu as pltpu
from jax.experimental.pallas import tpu_sc as plsc

N = 10000
E = 320000
D_IN = 128
D1 = 32
D2 = 16
N_CLASSES = 8

NW = 32
CH = 128
WPC = 80
NBUF = 4
E_PAD = NW * WPC * CH
N_PAD = 10240
ROWS_PER_SUB = N_PAD // 16

def _make_segsum(D, mesh):
    @functools.partial(
        pl.kernel,
        out_type=jax.ShapeDtypeStruct((2, N_PAD, D), jnp.float32),
        scratch_types=[
            pltpu.VMEM((WPC, CH), jnp.int32),
            pltpu.VMEM((WPC, CH), jnp.int32),
            pltpu.VMEM((NBUF, CH, D), jnp.float32),
            pltpu.VMEM_SHARED((N_PAD, D), jnp.float32),
            pltpu.VMEM_SHARED((N_PAD, D), jnp.float32),
            [pltpu.SemaphoreType.DMA] * NBUF,
        ],
        mesh=mesh,
        compiler_params=pltpu.CompilerParams(use_tc_tiling_on_sc=False),
    )
    def segsum(g_hbm, edges_hbm, zeros_hbm, out_hbm,
               idx_s, idx_d, rows, acc, g_sh, sems):
        c = lax.axis_index("c")
        s = lax.axis_index("s")
        w = c * 16 + s
        base = s * ROWS_PER_SUB

        def gather(j, b, src):
            return pltpu.make_async_copy(src.at[idx_s.at[j]], rows.at[b],
                                         sems[b])

        pltpu.sync_copy(edges_hbm.at[0, pl.ds(w * WPC, WPC)], idx_s)
        pltpu.sync_copy(edges_hbm.at[1, pl.ds(w * WPC, WPC)], idx_d)
        pltpu.sync_copy(g_hbm.at[pl.ds(base, ROWS_PER_SUB)],
                        g_sh.at[pl.ds(base, ROWS_PER_SUB)])
        pltpu.sync_copy(zeros_hbm.at[pl.ds(base, ROWS_PER_SUB)],
                        acc.at[pl.ds(base, ROWS_PER_SUB)])
        plsc.subcore_barrier()

        @pl.loop(0, WPC, step=NBUF)
        def _ring(jj):
            descs = [gather(jj + b, b, g_sh if b % 2 else g_hbm)
                     for b in range(NBUF)]
            for d in descs:
                d.start()
            for b in range(NBUF):
                descs[b].wait()
                pltpu.sync_copy(rows.at[b], acc.at[idx_d.at[jj + b]], add=True)

        plsc.subcore_barrier()
        pltpu.sync_copy(acc.at[pl.ds(base, ROWS_PER_SUB)],
                        out_hbm.at[c, pl.ds(base, ROWS_PER_SUB)])

    return segsum


@functools.lru_cache(maxsize=None)
def _sc_kernels():
    mesh = plsc.VectorSubcoreMesh(core_axis_name="c", subcore_axis_name="s")

    @functools.partial(
        pl.kernel,
        out_type=jax.ShapeDtypeStruct((2, N_PAD, D2), jnp.float32),
        scratch_types=[
            pltpu.VMEM((WPC, CH), jnp.int32),
            pltpu.VMEM((CH, D2), jnp.float32),
            pltpu.VMEM_SHARED((N_PAD, D2), jnp.float32),
        ],
        mesh=mesh,
        compiler_params=pltpu.CompilerParams(use_tc_tiling_on_sc=False),
    )
    def degree(edges_hbm, ones_hbm, zeros_hbm, out_hbm, idx_d, ones_v, acc):
        c = lax.axis_index("c")
        s = lax.axis_index("s")
        w = c * 16 + s
        base = s * ROWS_PER_SUB
        pltpu.sync_copy(edges_hbm.at[1, pl.ds(w * WPC, WPC)], idx_d)
        pltpu.sync_copy(ones_hbm, ones_v)
        pltpu.sync_copy(zeros_hbm.at[pl.ds(base, ROWS_PER_SUB)],
                        acc.at[pl.ds(base, ROWS_PER_SUB)])
        plsc.subcore_barrier()

        def step(j, carry):
            pltpu.sync_copy(ones_v, acc.at[idx_d.at[j]], add=True)
            return carry

        lax.fori_loop(0, WPC, step, 0)
        plsc.subcore_barrier()
        pltpu.sync_copy(acc.at[pl.ds(base, ROWS_PER_SUB)],
                        out_hbm.at[c, pl.ds(base, ROWS_PER_SUB)])

    return degree, _make_segsum(D1, mesh), _make_segsum(D2, mesh)


def _dot(a, b):
    return jnp.dot(a, b, preferred_element_type=jnp.float32)


def _tca_body(x_ref, w1_ref, h_ref):
    h_ref[:N, :] = _dot(x_ref[...], w1_ref[...])
    h_ref[N:, :] = jnp.zeros((N_PAD - N, D1), jnp.float32)


def _tcb_body(h_ref, degp_ref, g1_ref, dinv_ref):
    deg = degp_ref[0] + degp_ref[1] + 1.0
    dinv = lax.rsqrt(deg[:, :1])
    g1_ref[...] = h_ref[...] * dinv
    dinv_ref[...] = jnp.broadcast_to(dinv, (N_PAD, 8))


def _tc2_body(acc_ref, g1_ref, dinv_ref, b1_ref, w2_ref, g2_ref):
    dinv = dinv_ref[:N, :1]
    a = acc_ref[0, :N, :] + acc_ref[1, :N, :] + g1_ref[:N, :]
    out1 = jnp.maximum(dinv * a + b1_ref[...], 0.0)
    g2_ref[:N, :] = _dot(out1, w2_ref[...]) * dinv
    g2_ref[N:, :] = jnp.zeros((N_PAD - N, D2), jnp.float32)


def _tc3_body(acc_ref, g2_ref, dinv_ref, b2_ref, gwt_ref, gb_ref, lw_ref, out_ref):
    dinv = dinv_ref[:N, :1]
    a = acc_ref[0, :N, :] + acc_ref[1, :N, :] + g2_ref[:N, :]
    out2 = dinv * a + b2_ref[...]
    gates = jax.nn.sigmoid(_dot(out2, gwt_ref[...]) + gb_ref[...])
    out_ref[...] = _dot(gates, lw_ref[...])


def kernel(x, edge_index, W1, b1, W2, b2, gate_weights, gate_bias, leaf_weights):
    _degree, _segsum32, _segsum16 = _sc_kernels()
    ei = edge_index.astype(jnp.int32)
    edges = jnp.pad(ei, ((0, 0), (0, E_PAD - E)),
                    constant_values=N).reshape(2, NW * WPC, CH)

    zeros16 = jnp.zeros((N_PAD, D2), jnp.float32)
    zeros32 = jnp.zeros((N_PAD, D1), jnp.float32)
    ones_ch = jnp.ones((CH, D2), jnp.float32)

    degp = _degree(edges, ones_ch, zeros16)

    h1 = pl.pallas_call(
        _tca_body,
        out_shape=jax.ShapeDtypeStruct((N_PAD, D1), jnp.float32),
    )(x, W1)

    g1, dinv = pl.pallas_call(
        _tcb_body,
        out_shape=[jax.ShapeDtypeStruct((N_PAD, D1), jnp.float32),
                   jax.ShapeDtypeStruct((N_PAD, 8), jnp.float32)],
    )(h1, degp)

    acc1 = _segsum32(g1, edges, zeros32)

    g2 = pl.pallas_call(
        _tc2_body,
        out_shape=jax.ShapeDtypeStruct((N_PAD, D2), jnp.float32),
    )(acc1, g1, dinv, b1.reshape(1, D1), W2)

    acc2 = _segsum16(g2, edges, zeros16)

    out = pl.pallas_call(
        _tc3_body,
        out_shape=jax.ShapeDtypeStruct((N, N_CLASSES), jnp.float32),
    )(acc2, g2, dinv, b2.reshape(1, D2), gate_weights.T, gate_bias.reshape(1, D2),
      leaf_weights)

    return out


# speedup vs baseline: 1.3085x; 1.3085x over previous
import functools

import jax
import jax.numpy as jnp
from jax import lax
from jax.experimental import pallas as pl
from jax.experimental.pallas import tpu as pltpu
from jax.experimental.pallas import tpu_sc as plsc

N = 10000
E = 320000
D_IN = 128
D1 = 32
D2 = 16
N_CLASSES = 8

NW = 32
CH = 128
WPC = 80
NBUF = 4
E_PAD = NW * WPC * CH
N_PAD = 10240
ROWS_PER_SUB = N_PAD // 16

def _make_segsum(D, mesh):
    @functools.partial(
        pl.kernel,
        out_type=jax.ShapeDtypeStruct((2, N_PAD, D), jnp.float32),
        scratch_types=[
            pltpu.VMEM((WPC, CH), jnp.int32),
            pltpu.VMEM((WPC, CH), jnp.int32),
            pltpu.VMEM((NBUF, CH, D), jnp.float32),
            pltpu.VMEM_SHARED((N_PAD, D), jnp.float32),
            pltpu.VMEM_SHARED((N_PAD, D), jnp.float32),
            [pltpu.SemaphoreType.DMA] * NBUF,
        ],
        mesh=mesh,
        compiler_params=pltpu.CompilerParams(use_tc_tiling_on_sc=False),
    )
    def segsum(g_hbm, edges_hbm, zeros_hbm, out_hbm,
               idx_s, idx_d, rows, acc, g_sh, sems):
        c = lax.axis_index("c")
        s = lax.axis_index("s")
        w = c * 16 + s
        base = s * ROWS_PER_SUB

        def gather(j, b, src):
            return pltpu.make_async_copy(src.at[idx_s.at[j]], rows.at[b],
                                         sems[b])

        pltpu.sync_copy(edges_hbm.at[0, pl.ds(w * WPC, WPC)], idx_s)
        pltpu.sync_copy(edges_hbm.at[1, pl.ds(w * WPC, WPC)], idx_d)
        pltpu.sync_copy(g_hbm.at[pl.ds(base, ROWS_PER_SUB)],
                        g_sh.at[pl.ds(base, ROWS_PER_SUB)])
        pltpu.sync_copy(zeros_hbm.at[pl.ds(base, ROWS_PER_SUB)],
                        acc.at[pl.ds(base, ROWS_PER_SUB)])
        plsc.subcore_barrier()

        @pl.loop(0, WPC, step=NBUF)
        def _ring(jj):
            descs = [gather(jj + b, b, g_sh) for b in range(NBUF)]
            for d in descs:
                d.start()
            for b in range(NBUF):
                descs[b].wait()
                pltpu.sync_copy(rows.at[b], acc.at[idx_d.at[jj + b]], add=True)

        plsc.subcore_barrier()
        pltpu.sync_copy(acc.at[pl.ds(base, ROWS_PER_SUB)],
                        out_hbm.at[c, pl.ds(base, ROWS_PER_SUB)])

    return segsum


@functools.lru_cache(maxsize=None)
def _sc_kernels():
    mesh = plsc.VectorSubcoreMesh(core_axis_name="c", subcore_axis_name="s")

    @functools.partial(
        pl.kernel,
        out_type=jax.ShapeDtypeStruct((2, N_PAD, D2), jnp.float32),
        scratch_types=[
            pltpu.VMEM((WPC, CH), jnp.int32),
            pltpu.VMEM((CH, D2), jnp.float32),
            pltpu.VMEM_SHARED((N_PAD, D2), jnp.float32),
            [pltpu.SemaphoreType.DMA] * NBUF,
        ],
        mesh=mesh,
        compiler_params=pltpu.CompilerParams(use_tc_tiling_on_sc=False),
    )
    def degree(edges_hbm, ones_hbm, zeros_hbm, out_hbm, idx_d, ones_v, acc,
               sems):
        c = lax.axis_index("c")
        s = lax.axis_index("s")
        w = c * 16 + s
        base = s * ROWS_PER_SUB
        pltpu.sync_copy(edges_hbm.at[1, pl.ds(w * WPC, WPC)], idx_d)
        pltpu.sync_copy(ones_hbm, ones_v)
        pltpu.sync_copy(zeros_hbm.at[pl.ds(base, ROWS_PER_SUB)],
                        acc.at[pl.ds(base, ROWS_PER_SUB)])
        plsc.subcore_barrier()

        @pl.loop(0, WPC, step=NBUF)
        def _groups(jj):
            descs = [pltpu.async_copy(ones_v, acc.at[idx_d.at[jj + b]],
                                      sems[b], add=True)
                     for b in range(NBUF)]
            for d in descs:
                d.wait()

        plsc.subcore_barrier()
        pltpu.sync_copy(acc.at[pl.ds(base, ROWS_PER_SUB)],
                        out_hbm.at[c, pl.ds(base, ROWS_PER_SUB)])

    return degree, _make_segsum(D1, mesh), _make_segsum(D2, mesh)


def _dot(a, b):
    return jnp.dot(a, b, preferred_element_type=jnp.float32)


def _tca_body(x_ref, w1_ref, h_ref):
    h_ref[:N, :] = _dot(x_ref[...], w1_ref[...])
    h_ref[N:, :] = jnp.zeros((N_PAD - N, D1), jnp.float32)


def _tcb_body(h_ref, degp_ref, g1_ref, dinv_ref):
    deg = degp_ref[0] + degp_ref[1] + 1.0
    dinv = lax.rsqrt(deg[:, :1])
    g1_ref[...] = h_ref[...] * dinv
    dinv_ref[...] = jnp.broadcast_to(dinv, (N_PAD, 8))


def _tc2_body(acc_ref, g1_ref, dinv_ref, b1_ref, w2_ref, g2_ref):
    dinv = dinv_ref[:N, :1]
    a = acc_ref[0, :N, :] + acc_ref[1, :N, :] + g1_ref[:N, :]
    out1 = jnp.maximum(dinv * a + b1_ref[...], 0.0)
    g2_ref[:N, :] = _dot(out1, w2_ref[...]) * dinv
    g2_ref[N:, :] = jnp.zeros((N_PAD - N, D2), jnp.float32)


def _tc3_body(acc_ref, g2_ref, dinv_ref, b2_ref, gwt_ref, gb_ref, lw_ref, out_ref):
    dinv = dinv_ref[:N, :1]
    a = acc_ref[0, :N, :] + acc_ref[1, :N, :] + g2_ref[:N, :]
    out2 = dinv * a + b2_ref[...]
    gates = jax.nn.sigmoid(_dot(out2, gwt_ref[...]) + gb_ref[...])
    out_ref[...] = _dot(gates, lw_ref[...])


def kernel(x, edge_index, W1, b1, W2, b2, gate_weights, gate_bias, leaf_weights):
    _degree, _segsum32, _segsum16 = _sc_kernels()
    ei = edge_index.astype(jnp.int32)
    edges = jnp.pad(ei, ((0, 0), (0, E_PAD - E)),
                    constant_values=N).reshape(2, NW * WPC, CH)

    zeros16 = jnp.zeros((N_PAD, D2), jnp.float32)
    zeros32 = jnp.zeros((N_PAD, D1), jnp.float32)
    ones_ch = jnp.ones((CH, D2), jnp.float32)

    degp = _degree(edges, ones_ch, zeros16)

    h1 = pl.pallas_call(
        _tca_body,
        out_shape=jax.ShapeDtypeStruct((N_PAD, D1), jnp.float32),
    )(x, W1)

    g1, dinv = pl.pallas_call(
        _tcb_body,
        out_shape=[jax.ShapeDtypeStruct((N_PAD, D1), jnp.float32),
                   jax.ShapeDtypeStruct((N_PAD, 8), jnp.float32)],
    )(h1, degp)

    acc1 = _segsum32(g1, edges, zeros32)

    g2 = pl.pallas_call(
        _tc2_body,
        out_shape=jax.ShapeDtypeStruct((N_PAD, D2), jnp.float32),
    )(acc1, g1, dinv, b1.reshape(1, D1), W2)

    acc2 = _segsum16(g2, edges, zeros16)

    out = pl.pallas_call(
        _tc3_body,
        out_shape=jax.ShapeDtypeStruct((N, N_CLASSES), jnp.float32),
    )(acc2, g2, dinv, b2.reshape(1, D2), gate_weights.T, gate_bias.reshape(1, D2),
      leaf_weights)

    return out


# speedup vs baseline: 1.3431x; 1.0265x over previous
import functools

import jax
import jax.numpy as jnp
from jax import lax
from jax.experimental import pallas as pl
from jax.experimental.pallas import tpu as pltpu
from jax.experimental.pallas import tpu_sc as plsc

N = 10000
E = 320000
D_IN = 128
D1 = 32
D2 = 16
N_CLASSES = 8

NW = 32
CH = 128
WPC = 80
NBUF = 8
E_PAD = NW * WPC * CH
N_PAD = 10240
ROWS_PER_SUB = N_PAD // 16

def _make_segsum(D, mesh):
    @functools.partial(
        pl.kernel,
        out_type=jax.ShapeDtypeStruct((2, N_PAD, D), jnp.float32),
        scratch_types=[
            pltpu.VMEM((WPC, CH), jnp.int32),
            pltpu.VMEM((WPC, CH), jnp.int32),
            pltpu.VMEM((NBUF, CH, D), jnp.float32),
            pltpu.VMEM_SHARED((N_PAD, D), jnp.float32),
            pltpu.VMEM_SHARED((N_PAD, D), jnp.float32),
            [pltpu.SemaphoreType.DMA] * NBUF,
        ],
        mesh=mesh,
        compiler_params=pltpu.CompilerParams(use_tc_tiling_on_sc=False),
    )
    def segsum(g_hbm, edges_hbm, zeros_hbm, out_hbm,
               idx_s, idx_d, rows, acc, g_sh, sems):
        c = lax.axis_index("c")
        s = lax.axis_index("s")
        w = c * 16 + s
        base = s * ROWS_PER_SUB

        def gather(j, b, src):
            return pltpu.make_async_copy(src.at[idx_s.at[j]], rows.at[b],
                                         sems[b])

        pltpu.sync_copy(edges_hbm.at[0, pl.ds(w * WPC, WPC)], idx_s)
        pltpu.sync_copy(edges_hbm.at[1, pl.ds(w * WPC, WPC)], idx_d)
        pltpu.sync_copy(g_hbm.at[pl.ds(base, ROWS_PER_SUB)],
                        g_sh.at[pl.ds(base, ROWS_PER_SUB)])
        pltpu.sync_copy(zeros_hbm.at[pl.ds(base, ROWS_PER_SUB)],
                        acc.at[pl.ds(base, ROWS_PER_SUB)])
        plsc.subcore_barrier()

        @pl.loop(0, WPC, step=NBUF)
        def _ring(jj):
            descs = [gather(jj + b, b, g_sh) for b in range(NBUF)]
            for d in descs:
                d.start()
            for b in range(NBUF):
                descs[b].wait()
                pltpu.sync_copy(rows.at[b], acc.at[idx_d.at[jj + b]], add=True)

        plsc.subcore_barrier()
        pltpu.sync_copy(acc.at[pl.ds(base, ROWS_PER_SUB)],
                        out_hbm.at[c, pl.ds(base, ROWS_PER_SUB)])

    return segsum


@functools.lru_cache(maxsize=None)
def _sc_kernels():
    mesh = plsc.VectorSubcoreMesh(core_axis_name="c", subcore_axis_name="s")

    @functools.partial(
        pl.kernel,
        out_type=jax.ShapeDtypeStruct((2, N_PAD, D2), jnp.float32),
        scratch_types=[
            pltpu.VMEM((WPC, CH), jnp.int32),
            pltpu.VMEM((CH, D2), jnp.float32),
            pltpu.VMEM_SHARED((N_PAD, D2), jnp.float32),
        ],
        mesh=mesh,
        compiler_params=pltpu.CompilerParams(use_tc_tiling_on_sc=False),
    )
    def degree(edges_hbm, ones_hbm, zeros_hbm, out_hbm, idx_d, ones_v, acc):
        c = lax.axis_index("c")
        s = lax.axis_index("s")
        w = c * 16 + s
        base = s * ROWS_PER_SUB
        pltpu.sync_copy(edges_hbm.at[1, pl.ds(w * WPC, WPC)], idx_d)
        pltpu.sync_copy(ones_hbm, ones_v)
        pltpu.sync_copy(zeros_hbm.at[pl.ds(base, ROWS_PER_SUB)],
                        acc.at[pl.ds(base, ROWS_PER_SUB)])
        plsc.subcore_barrier()

        def step(j, carry):
            pltpu.sync_copy(ones_v, acc.at[idx_d.at[j]], add=True)
            return carry

        lax.fori_loop(0, WPC, step, 0)
        plsc.subcore_barrier()
        pltpu.sync_copy(acc.at[pl.ds(base, ROWS_PER_SUB)],
                        out_hbm.at[c, pl.ds(base, ROWS_PER_SUB)])

    return degree, _make_segsum(D1, mesh), _make_segsum(D2, mesh)


def _dot(a, b):
    return jnp.dot(a, b, preferred_element_type=jnp.float32)


def _tca_body(x_ref, w1_ref, h_ref):
    h_ref[:N, :] = _dot(x_ref[...], w1_ref[...])
    h_ref[N:, :] = jnp.zeros((N_PAD - N, D1), jnp.float32)


def _tcb_body(h_ref, degp_ref, g1_ref, dinv_ref):
    deg = degp_ref[0] + degp_ref[1] + 1.0
    dinv = lax.rsqrt(deg[:, :1])
    g1_ref[...] = h_ref[...] * dinv
    dinv_ref[...] = jnp.broadcast_to(dinv, (N_PAD, 8))


def _tc2_body(acc_ref, g1_ref, dinv_ref, b1_ref, w2_ref, g2_ref):
    dinv = dinv_ref[:N, :1]
    a = acc_ref[0, :N, :] + acc_ref[1, :N, :] + g1_ref[:N, :]
    out1 = jnp.maximum(dinv * a + b1_ref[...], 0.0)
    g2_ref[:N, :] = _dot(out1, w2_ref[...]) * dinv
    g2_ref[N:, :] = jnp.zeros((N_PAD - N, D2), jnp.float32)


def _tc3_body(acc_ref, g2_ref, dinv_ref, b2_ref, gwt_ref, gb_ref, lw_ref, out_ref):
    dinv = dinv_ref[:N, :1]
    a = acc_ref[0, :N, :] + acc_ref[1, :N, :] + g2_ref[:N, :]
    out2 = dinv * a + b2_ref[...]
    gates = jax.nn.sigmoid(_dot(out2, gwt_ref[...]) + gb_ref[...])
    out_ref[...] = _dot(gates, lw_ref[...])


def kernel(x, edge_index, W1, b1, W2, b2, gate_weights, gate_bias, leaf_weights):
    _degree, _segsum32, _segsum16 = _sc_kernels()
    ei = edge_index.astype(jnp.int32)
    edges = jnp.pad(ei, ((0, 0), (0, E_PAD - E)),
                    constant_values=N).reshape(2, NW * WPC, CH)

    zeros16 = jnp.zeros((N_PAD, D2), jnp.float32)
    zeros32 = jnp.zeros((N_PAD, D1), jnp.float32)
    ones_ch = jnp.ones((CH, D2), jnp.float32)

    degp = _degree(edges, ones_ch, zeros16)

    h1 = pl.pallas_call(
        _tca_body,
        out_shape=jax.ShapeDtypeStruct((N_PAD, D1), jnp.float32),
    )(x, W1)

    g1, dinv = pl.pallas_call(
        _tcb_body,
        out_shape=[jax.ShapeDtypeStruct((N_PAD, D1), jnp.float32),
                   jax.ShapeDtypeStruct((N_PAD, 8), jnp.float32)],
    )(h1, degp)

    acc1 = _segsum32(g1, edges, zeros32)

    g2 = pl.pallas_call(
        _tc2_body,
        out_shape=jax.ShapeDtypeStruct((N_PAD, D2), jnp.float32),
    )(acc1, g1, dinv, b1.reshape(1, D1), W2)

    acc2 = _segsum16(g2, edges, zeros16)

    out = pl.pallas_call(
        _tc3_body,
        out_shape=jax.ShapeDtypeStruct((N, N_CLASSES), jnp.float32),
    )(acc2, g2, dinv, b2.reshape(1, D2), gate_weights.T, gate_bias.reshape(1, D2),
      leaf_weights)

    return out


# speedup vs baseline: 1.3525x; 1.0070x over previous
import functools

import jax
import jax.numpy as jnp
from jax import lax
from jax.experimental import pallas as pl
from jax.experimental.pallas import tpu as pltpu
from jax.experimental.pallas import tpu_sc as plsc

N = 10000
E = 320000
D_IN = 128
D1 = 32
D2 = 16
N_CLASSES = 8

NW = 32
CH = 128
WPC = 80
NBUF = 8
E_PAD = NW * WPC * CH
N_PAD = 10240
ROWS_PER_SUB = N_PAD // 16

def _make_segsum(D, mesh):
    @functools.partial(
        pl.kernel,
        out_type=jax.ShapeDtypeStruct((2, N_PAD, D), jnp.float32),
        scratch_types=[
            pltpu.VMEM((WPC, CH), jnp.int32),
            pltpu.VMEM((WPC, CH), jnp.int32),
            pltpu.VMEM((NBUF, CH, D), jnp.float32),
            pltpu.VMEM_SHARED((N_PAD, D), jnp.float32),
            pltpu.VMEM_SHARED((N_PAD, D), jnp.float32),
            [pltpu.SemaphoreType.DMA] * NBUF,
        ],
        mesh=mesh,
        compiler_params=pltpu.CompilerParams(use_tc_tiling_on_sc=False),
    )
    def segsum(g_hbm, edges_hbm, zeros_hbm, out_hbm,
               idx_s, idx_d, rows, acc, g_sh, sems):
        c = lax.axis_index("c")
        s = lax.axis_index("s")
        w = c * 16 + s
        base = s * ROWS_PER_SUB

        def gather(j, b, src):
            return pltpu.make_async_copy(src.at[idx_s.at[j]], rows.at[b],
                                         sems[b])

        pltpu.sync_copy(edges_hbm.at[0, pl.ds(w * WPC, WPC)], idx_s)
        pltpu.sync_copy(edges_hbm.at[1, pl.ds(w * WPC, WPC)], idx_d)
        pltpu.sync_copy(g_hbm.at[pl.ds(base, ROWS_PER_SUB)],
                        g_sh.at[pl.ds(base, ROWS_PER_SUB)])
        pltpu.sync_copy(zeros_hbm.at[pl.ds(base, ROWS_PER_SUB)],
                        acc.at[pl.ds(base, ROWS_PER_SUB)])
        plsc.subcore_barrier()

        @pl.loop(0, WPC, step=NBUF)
        def _ring(jj):
            descs = [gather(jj + b, b, g_sh) for b in range(NBUF)]
            for d in descs:
                d.start()
            for b in range(NBUF):
                descs[b].wait()
                pltpu.sync_copy(rows.at[b], acc.at[idx_d.at[jj + b]], add=True)

        plsc.subcore_barrier()
        pltpu.sync_copy(acc.at[pl.ds(base, ROWS_PER_SUB)],
                        out_hbm.at[c, pl.ds(base, ROWS_PER_SUB)])

    return segsum


@functools.lru_cache(maxsize=None)
def _sc_kernels():
    mesh = plsc.VectorSubcoreMesh(core_axis_name="c", subcore_axis_name="s")

    @functools.partial(
        pl.kernel,
        out_type=jax.ShapeDtypeStruct((2, N_PAD, D2), jnp.float32),
        scratch_types=[
            pltpu.VMEM((WPC, CH), jnp.int32),
            pltpu.VMEM((CH, D2), jnp.float32),
            pltpu.VMEM_SHARED((N_PAD, D2), jnp.float32),
        ],
        mesh=mesh,
        compiler_params=pltpu.CompilerParams(use_tc_tiling_on_sc=False),
    )
    def degree(edges_hbm, ones_hbm, zeros_hbm, out_hbm, idx_d, ones_v, acc):
        c = lax.axis_index("c")
        s = lax.axis_index("s")
        w = c * 16 + s
        base = s * ROWS_PER_SUB
        pltpu.sync_copy(edges_hbm.at[1, pl.ds(w * WPC, WPC)], idx_d)
        pltpu.sync_copy(ones_hbm, ones_v)
        pltpu.sync_copy(zeros_hbm.at[pl.ds(base, ROWS_PER_SUB)],
                        acc.at[pl.ds(base, ROWS_PER_SUB)])
        plsc.subcore_barrier()

        def step(j, carry):
            pltpu.sync_copy(ones_v, acc.at[idx_d.at[j]], add=True)
            return carry

        lax.fori_loop(0, WPC, step, 0)
        plsc.subcore_barrier()
        pltpu.sync_copy(acc.at[pl.ds(base, ROWS_PER_SUB)],
                        out_hbm.at[c, pl.ds(base, ROWS_PER_SUB)])

    return degree, _make_segsum(D1, mesh), _make_segsum(D2, mesh)


def _dot(a, b):
    return jnp.dot(a, b, preferred_element_type=jnp.float32)


BLK = 2560
_GRID = N_PAD // BLK


def _row_mask(i, val):
    rid = i * BLK + lax.broadcasted_iota(jnp.int32, (BLK, 1), 0)
    return jnp.where(rid < N, val, 0.0)


def _tca_body(x_ref, w1_ref, h_ref):
    i = pl.program_id(0)
    h_ref[...] = _row_mask(i, _dot(x_ref[...], w1_ref[...]))


def _tcb_body(h_ref, degp_ref, g1_ref, dinv_ref):
    deg = degp_ref[0] + degp_ref[1] + 1.0
    dinv = lax.rsqrt(deg[:, :1])
    g1_ref[...] = h_ref[...] * dinv
    dinv_ref[...] = jnp.broadcast_to(dinv, (BLK, 8))


def _tc2_body(acc_ref, g1_ref, dinv_ref, b1_ref, w2_ref, g2_ref):
    i = pl.program_id(0)
    dinv = dinv_ref[:, :1]
    a = acc_ref[0] + acc_ref[1] + g1_ref[...]
    out1 = jnp.maximum(dinv * a + b1_ref[...], 0.0)
    g2_ref[...] = _row_mask(i, _dot(out1, w2_ref[...]) * dinv)


def _tc3_body(acc_ref, g2_ref, dinv_ref, b2_ref, gwt_ref, gb_ref, lw_ref, out_ref):
    dinv = dinv_ref[:, :1]
    a = acc_ref[0] + acc_ref[1] + g2_ref[...]
    out2 = dinv * a + b2_ref[...]
    gates = jax.nn.sigmoid(_dot(out2, gwt_ref[...]) + gb_ref[...])
    out_ref[...] = _dot(gates, lw_ref[...])


def _rows(bs):
    return pl.BlockSpec(bs, lambda i: (0,) * (len(bs) - 2) + (i, 0))


def _full(bs):
    return pl.BlockSpec(bs, lambda i: (0,) * len(bs))


def kernel(x, edge_index, W1, b1, W2, b2, gate_weights, gate_bias, leaf_weights):
    _degree, _segsum32, _segsum16 = _sc_kernels()
    ei = edge_index.astype(jnp.int32)
    edges = jnp.pad(ei, ((0, 0), (0, E_PAD - E)),
                    constant_values=N).reshape(2, NW * WPC, CH)

    zeros16 = jnp.zeros((N_PAD, D2), jnp.float32)
    zeros32 = jnp.zeros((N_PAD, D1), jnp.float32)
    ones_ch = jnp.ones((CH, D2), jnp.float32)

    degp = _degree(edges, ones_ch, zeros16)

    h1 = pl.pallas_call(
        _tca_body,
        grid=(_GRID,),
        in_specs=[_rows((BLK, D_IN)), _full((D_IN, D1))],
        out_specs=_rows((BLK, D1)),
        out_shape=jax.ShapeDtypeStruct((N_PAD, D1), jnp.float32),
    )(x, W1)

    g1, dinv = pl.pallas_call(
        _tcb_body,
        grid=(_GRID,),
        in_specs=[_rows((BLK, D1)), _rows((2, BLK, D2))],
        out_specs=[_rows((BLK, D1)), _rows((BLK, 8))],
        out_shape=[jax.ShapeDtypeStruct((N_PAD, D1), jnp.float32),
                   jax.ShapeDtypeStruct((N_PAD, 8), jnp.float32)],
    )(h1, degp)

    acc1 = _segsum32(g1, edges, zeros32)

    g2 = pl.pallas_call(
        _tc2_body,
        grid=(_GRID,),
        in_specs=[_rows((2, BLK, D1)), _rows((BLK, D1)), _rows((BLK, 8)),
                  _full((1, D1)), _full((D1, D2))],
        out_specs=_rows((BLK, D2)),
        out_shape=jax.ShapeDtypeStruct((N_PAD, D2), jnp.float32),
    )(acc1, g1, dinv, b1.reshape(1, D1), W2)

    acc2 = _segsum16(g2, edges, zeros16)

    out = pl.pallas_call(
        _tc3_body,
        grid=(_GRID,),
        in_specs=[_rows((2, BLK, D2)), _rows((BLK, D2)), _rows((BLK, 8)),
                  _full((1, D2)), _full((D2, D2)), _full((1, D2)),
                  _full((D2, N_CLASSES))],
        out_specs=_rows((BLK, N_CLASSES)),
        out_shape=jax.ShapeDtypeStruct((N, N_CLASSES), jnp.float32),
    )(acc2, g2, dinv, b2.reshape(1, D2), gate_weights.T, gate_bias.reshape(1, D2),
      leaf_weights)

    return out
